# Initial kernel scaffold; baseline (speedup 1.0000x reference)
import functools

import jax
import jax.numpy as jnp
from jax import lax
from jax.experimental import pallas as pl
from jax.experimental.pallas import tpu as pltpu
from jax.experimental.pallas import tpu_sc as plsc

F32 = jnp.float32

_N = 10000
_E = 320000
_D = 128
_W = 144
_NS = 16
_CH = 128
_EPT = 20480
_EPAD = _NS * _EPT
_NCHUNK = _EPT // _CH
_RPT = _N // _NS

_O = (0, 2 * _N + 8, 4 * _N + 16, 5 * _N + 24)


def _sc_body(ta, gidx, dstp, zrows, outf, acc, gi_v, di_v, rows_v, sem):
    c = lax.axis_index("c")
    s = lax.axis_index("s")
    tbase = s * _EPT
    for j in range(2):
        jj = 2 * c + j
        pltpu.sync_copy(zrows.at[pl.ds(s * _RPT, _RPT)],
                        acc.at[pl.ds(s * _RPT, _RPT)])
        plsc.subcore_barrier()

        def chunk(k, carry):
            goff = jj * _EPAD + tbase + k * _CH
            eoff = tbase + k * _CH
            pltpu.sync_copy(gidx.at[pl.ds(goff, _CH)], gi_v)
            pltpu.sync_copy(dstp.at[pl.ds(eoff, _CH)], di_v)
            pltpu.async_copy(ta.at[gi_v], rows_v, sem).wait()
            pltpu.sync_copy(rows_v, acc.at[di_v], add=True)
            return carry

        lax.fori_loop(0, _NCHUNK, chunk, 0)
        plsc.subcore_barrier()
        pltpu.sync_copy(acc.at[pl.ds(s * _RPT, _RPT)],
                        outf.at[pl.ds(jj * _N + s * _RPT, _RPT)])
        plsc.subcore_barrier()


_sc_agg = functools.partial(
    pl.kernel,
    mesh=plsc.VectorSubcoreMesh(core_axis_name="c", subcore_axis_name="s"),
    out_type=jax.ShapeDtypeStruct((4 * _N, _W), F32),
    scratch_types=[
        pltpu.VMEM_SHARED((_N, _W), F32),
        pltpu.VMEM((_CH,), jnp.int32),
        pltpu.VMEM((_CH,), jnp.int32),
        pltpu.VMEM((_CH, _W), F32),
        pltpu.SemaphoreType.DMA,
    ],
)(_sc_body)


_R = 1000


def _tc_body(a1, a4, a2, a3, xs, xu, wcs, wcu, wfs, wfu, bf, os_ref, ou_ref):
    inv = 1.0 / jnp.maximum(a2[:, _D:_D + 1], 1.0)
    s_in = jnp.concatenate([a1[:, :_D], a4[:, :_D]], axis=1) * inv
    u_in = jnp.concatenate([a3[:, :_D], a2[:, :_D]], axis=1) * inv
    hs = jnp.tanh(
        jnp.dot(s_in, wcs[...], preferred_element_type=F32) + xs[...])
    hu = jnp.maximum(
        jnp.dot(u_in, wcu[...], preferred_element_type=F32) + xu[...], 0.0)
    os_ref[...] = jnp.dot(jnp.concatenate([hs, hu], axis=1), wfs[...],
                          preferred_element_type=F32) + hs
    ou_ref[...] = jnp.dot(jnp.concatenate([jnp.abs(hs), hu], axis=1), wfu[...],
                          preferred_element_type=F32) + bf[...] + hu


def _acc_spec(job):
    base = job * (_N // _R)
    return pl.BlockSpec((_R, _W), lambda i: (base + i, 0))


_tc_epilogue = pl.pallas_call(
    _tc_body,
    grid=(_N // _R,),
    in_specs=[
        _acc_spec(0), _acc_spec(1), _acc_spec(2), _acc_spec(3),
        pl.BlockSpec((_R, _D), lambda i: (i, 0)),
        pl.BlockSpec((_R, _D), lambda i: (i, 0)),
        pl.BlockSpec((2 * _D, _D), lambda i: (0, 0)),
        pl.BlockSpec((2 * _D, _D), lambda i: (0, 0)),
        pl.BlockSpec((2 * _D, _D), lambda i: (0, 0)),
        pl.BlockSpec((2 * _D, _D), lambda i: (0, 0)),
        pl.BlockSpec((1, _D), lambda i: (0, 0)),
    ],
    out_specs=[
        pl.BlockSpec((_R, _D), lambda i: (i, 0)),
        pl.BlockSpec((_R, _D), lambda i: (i, 0)),
    ],
    out_shape=[
        jax.ShapeDtypeStruct((_N, _D), F32),
        jax.ShapeDtypeStruct((_N, _D), F32),
    ],
)


def kernel(x_signed, x_unsigned, edge_index, is_directed,
           W_ss, W_su, W_uu, W_us, Wf_s, Wf_u, bf_u):
    src = edge_index[0]
    dst = edge_index[1]
    dir_i = is_directed.astype(jnp.int32)

    zc = jnp.zeros((_N, 16), F32)
    onec = jnp.concatenate(
        [jnp.ones((_N, 1), F32), jnp.zeros((_N, 15), F32)], axis=1)
    zrow8 = jnp.zeros((8, _W), F32)

    t1 = jnp.concatenate([
        jnp.concatenate([x_signed, zc], axis=1),
        jnp.concatenate([-x_signed, zc], axis=1),
        zrow8,
    ], axis=0)
    t4 = jnp.concatenate([
        jnp.concatenate([x_unsigned, zc], axis=1),
        jnp.concatenate([-x_unsigned, zc], axis=1),
        zrow8,
    ], axis=0)
    t2 = jnp.concatenate(
        [jnp.concatenate([jnp.abs(x_signed), onec], axis=1), zrow8], axis=0)
    t3 = jnp.concatenate(
        [jnp.concatenate([x_unsigned, zc], axis=1), zrow8], axis=0)
    ta = jnp.concatenate([t1, t4, t2, t3], axis=0)

    npad = _EPAD - _E
    g_sd = jnp.concatenate(
        [src + _N * dir_i, jnp.full((npad,), 2 * _N, jnp.int32)])
    g_u = jnp.concatenate([src, jnp.full((npad,), _N, jnp.int32)])
    gidx = jnp.concatenate([
        g_sd + _O[0], g_sd + _O[1], g_u + _O[2], g_u + _O[3]])
    dstp = jnp.concatenate([dst, jnp.zeros((npad,), jnp.int32)])
    zrows = jnp.zeros((_N, _W), F32)

    outf = _sc_agg(ta, gidx, dstp, zrows)

    new_s, new_u = _tc_epilogue(
        outf, outf, outf, outf, x_signed, x_unsigned,
        jnp.concatenate([W_ss, W_us], axis=0),
        jnp.concatenate([W_uu, W_su], axis=0),
        Wf_s, Wf_u, bf_u.reshape(1, _D))
    return (new_s, new_u)


# speedup vs baseline: 2.3160x; 2.3160x over previous
import functools

import jax
import jax.numpy as jnp
from jax import lax
from jax.experimental import pallas as pl
from jax.experimental.pallas import tpu as pltpu
from jax.experimental.pallas import tpu_sc as plsc

F32 = jnp.float32

_N = 10000
_E = 320000
_D = 128
_NS = 16
_CH = 128
_EPT = 20480
_EPAD = _NS * _EPT
_NCHUNK = _EPT // _CH
_EPT2 = _EPAD // 32
_NCHUNK2 = _EPT2 // _CH
_NPAD = 10240
_RPT = _NPAD // _NS

_O = (0, 2 * _N + 8, 4 * _N + 16, 5 * _N + 24)


def _sc_body(ta, gidx, dstp, zrows, ones_h, outf, acc, gi_v, di_v, rows_v,
             ones_v, sem):
    c = lax.axis_index("c")
    s = lax.axis_index("s")
    pltpu.sync_copy(ones_h, ones_v)
    for j in range(3):
        jj = 2 * c + j if j < 2 else 4 + c
        pltpu.sync_copy(zrows.at[pl.ds(s * _RPT, _RPT)],
                        acc.at[pl.ds(s * _RPT, _RPT)])
        plsc.subcore_barrier()

        if j < 2:
            tbase = s * _EPT

            def chunk(k, carry):
                goff = jj * _EPAD + tbase + k * _CH
                eoff = tbase + k * _CH
                pltpu.sync_copy(gidx.at[pl.ds(goff, _CH)], gi_v)
                pltpu.sync_copy(dstp.at[pl.ds(eoff, _CH)], di_v)
                pltpu.async_copy(ta.at[gi_v], rows_v, sem).wait()
                pltpu.sync_copy(rows_v, acc.at[di_v], add=True)
                return carry

            lax.fori_loop(0, _NCHUNK, chunk, 0)
        else:
            dbase = c * (_EPAD // 2) + s * _EPT2

            def dchunk(k, carry):
                eoff = dbase + k * _CH
                pltpu.sync_copy(dstp.at[pl.ds(eoff, _CH)], di_v)
                pltpu.sync_copy(ones_v, acc.at[di_v], add=True)
                return carry

            lax.fori_loop(0, _NCHUNK2, dchunk, 0)

        plsc.subcore_barrier()
        pltpu.sync_copy(acc.at[pl.ds(s * _RPT, _RPT)],
                        outf.at[jj, pl.ds(s * _RPT, _RPT)])
        plsc.subcore_barrier()


def _make_sc_agg():
    return functools.partial(
        pl.kernel,
        mesh=plsc.VectorSubcoreMesh(core_axis_name="c", subcore_axis_name="s"),
        out_type=jax.ShapeDtypeStruct((6, _NPAD, _D), F32),
        scratch_types=[
            pltpu.VMEM_SHARED((_NPAD, _D), F32),
            pltpu.VMEM((_CH,), jnp.int32),
            pltpu.VMEM((_CH,), jnp.int32),
            pltpu.VMEM((_CH, _D), F32),
            pltpu.VMEM((_CH, _D), F32),
            pltpu.SemaphoreType.DMA,
        ],
    )(_sc_body)


_R = 1000


def _tc_body(a1_3, a4_3, a2_3, a3_3, d0_3, d1_3, xs, xu, wcs, wcu, wfs, wfu,
             bf, os_ref, ou_ref):
    a1, a4, a2, a3 = a1_3[0], a4_3[0], a2_3[0], a3_3[0]
    deg = d0_3[0, :, :1] + d1_3[0, :, :1]
    inv = 1.0 / jnp.maximum(deg, 1.0)
    s_in = jnp.concatenate([a1, a4], axis=1) * inv
    u_in = jnp.concatenate([a3, a2], axis=1) * inv
    hs = jnp.tanh(
        jnp.dot(s_in, wcs[...], preferred_element_type=F32) + xs[...])
    hu = jnp.maximum(
        jnp.dot(u_in, wcu[...], preferred_element_type=F32) + xu[...], 0.0)
    os_ref[...] = jnp.dot(jnp.concatenate([hs, hu], axis=1), wfs[...],
                          preferred_element_type=F32) + hs
    ou_ref[...] = jnp.dot(jnp.concatenate([jnp.abs(hs), hu], axis=1), wfu[...],
                          preferred_element_type=F32) + bf[...] + hu


def _acc_spec(job):
    return pl.BlockSpec((1, _R, _D), lambda i: (job, i, 0))


_tc_epilogue = pl.pallas_call(
    _tc_body,
    grid=(_N // _R,),
    in_specs=[
        _acc_spec(0), _acc_spec(1), _acc_spec(2), _acc_spec(3),
        _acc_spec(4), _acc_spec(5),
        pl.BlockSpec((_R, _D), lambda i: (i, 0)),
        pl.BlockSpec((_R, _D), lambda i: (i, 0)),
        pl.BlockSpec((2 * _D, _D), lambda i: (0, 0)),
        pl.BlockSpec((2 * _D, _D), lambda i: (0, 0)),
        pl.BlockSpec((2 * _D, _D), lambda i: (0, 0)),
        pl.BlockSpec((2 * _D, _D), lambda i: (0, 0)),
        pl.BlockSpec((1, _D), lambda i: (0, 0)),
    ],
    out_specs=[
        pl.BlockSpec((_R, _D), lambda i: (i, 0)),
        pl.BlockSpec((_R, _D), lambda i: (i, 0)),
    ],
    out_shape=[
        jax.ShapeDtypeStruct((_N, _D), F32),
        jax.ShapeDtypeStruct((_N, _D), F32),
    ],
)


def kernel(x_signed, x_unsigned, edge_index, is_directed,
           W_ss, W_su, W_uu, W_us, Wf_s, Wf_u, bf_u):
    src = edge_index[0]
    dst = edge_index[1]
    dir_i = is_directed.astype(jnp.int32)

    zrow8 = jnp.zeros((8, _D), F32)
    t1 = jnp.concatenate([x_signed, -x_signed, zrow8], axis=0)
    t4 = jnp.concatenate([x_unsigned, -x_unsigned, zrow8], axis=0)
    t2 = jnp.concatenate([jnp.abs(x_signed), zrow8], axis=0)
    t3 = jnp.concatenate([x_unsigned, zrow8], axis=0)
    ta = jnp.concatenate([t1, t4, t2, t3], axis=0)

    npad = _EPAD - _E
    g_sd = jnp.concatenate(
        [src + _N * dir_i, jnp.full((npad,), 2 * _N, jnp.int32)])
    g_u = jnp.concatenate([src, jnp.full((npad,), _N, jnp.int32)])
    gidx = jnp.concatenate([
        g_sd + _O[0], g_sd + _O[1], g_u + _O[2], g_u + _O[3]])
    dstp = jnp.concatenate([dst, jnp.full((npad,), _N, jnp.int32)])
    zrows = jnp.zeros((_NPAD, _D), F32)
    ones_h = jnp.ones((_CH, _D), F32)

    outf = _make_sc_agg()(ta, gidx, dstp, zrows, ones_h)

    new_s, new_u = _tc_epilogue(
        outf, outf, outf, outf, outf, outf, x_signed, x_unsigned,
        jnp.concatenate([W_ss, W_us], axis=0),
        jnp.concatenate([W_uu, W_su], axis=0),
        Wf_s, Wf_u, bf_u.reshape(1, _D))
    return (new_s, new_u)


# speedup vs baseline: 3.2418x; 1.3998x over previous
import functools

import jax
import jax.numpy as jnp
from jax import lax
from jax.experimental import pallas as pl
from jax.experimental.pallas import tpu as pltpu
from jax.experimental.pallas import tpu_sc as plsc

F32 = jnp.float32

_N = 10000
_E = 320000
_D = 128
_NS = 16
_CH = 128
_EPT = 20480
_EPAD = _NS * _EPT
_NCHUNK = _EPT // _CH
_EPT2 = _EPAD // 32
_NCHUNK2 = _EPT2 // _CH
_G = 16
_NPAD = 10240
_RPT = _NPAD // _NS

_O = (0, 2 * _N + 8, 4 * _N + 16, 5 * _N + 24)


def _sc_body(ta, gidx4, dst2, zrows, ones_h, outf, acc, giv, div, rows,
             sg0, sg1, ss0, ss1):
    c = lax.axis_index("c")
    s = lax.axis_index("s")
    sg = (sg0, sg1)
    ss = (ss0, ss1)

    def drain(sem, b):
        pltpu.make_async_copy(ta.at[pl.ds(0, _CH)], rows.at[b], sem).wait()

    for j in range(3):
        jj = 2 * c + j if j < 2 else 4 + c
        pltpu.sync_copy(zrows.at[pl.ds(s * _RPT, _RPT)],
                        acc.at[pl.ds(s * _RPT, _RPT)])
        plsc.subcore_barrier()

        if j < 2:
            def group(grp, carry):
                pltpu.sync_copy(gidx4.at[jj, s, pl.ds(grp * _G, _G)], giv)
                pltpu.sync_copy(dst2.at[s, pl.ds(grp * _G, _G)], div)
                pltpu.async_copy(ta.at[giv.at[0]], rows.at[0], sg[0])
                for i in range(_G):
                    b = i % 2
                    nb = 1 - b
                    if i < _G - 1:
                        if i >= 1:
                            drain(ss[nb], nb)
                        pltpu.async_copy(ta.at[giv.at[i + 1]], rows.at[nb],
                                         sg[nb])
                    drain(sg[b], b)
                    pltpu.async_copy(rows.at[b], acc.at[div.at[i]], ss[b],
                                     add=True)
                drain(ss[0], 0)
                drain(ss[1], 1)
                return carry

            lax.fori_loop(0, _NCHUNK // _G, group, 0)
        else:
            wid = c * _NS + s
            pltpu.sync_copy(ones_h, rows.at[0])

            def dgroup(grp, carry):
                pltpu.sync_copy(
                    dst2.at[wid // 2,
                            pl.ds((wid % 2) * _NCHUNK2 + grp * _G, _G)],
                    div)
                for i in range(_G):
                    b = i % 2
                    if i >= 2:
                        drain(ss[b], b)
                    pltpu.async_copy(rows.at[0], acc.at[div.at[i]], ss[b],
                                     add=True)
                drain(ss[0], 0)
                drain(ss[1], 1)
                return carry

            lax.fori_loop(0, _NCHUNK2 // _G, dgroup, 0)

        plsc.subcore_barrier()
        pltpu.sync_copy(acc.at[pl.ds(s * _RPT, _RPT)],
                        outf.at[jj, pl.ds(s * _RPT, _RPT)])
        plsc.subcore_barrier()


def _make_sc_agg():
    return functools.partial(
        pl.kernel,
        mesh=plsc.VectorSubcoreMesh(core_axis_name="c", subcore_axis_name="s"),
        out_type=jax.ShapeDtypeStruct((6, _NPAD, _D), F32),
        scratch_types=[
            pltpu.VMEM_SHARED((_NPAD, _D), F32),
            pltpu.VMEM((_G, _CH), jnp.int32),
            pltpu.VMEM((_G, _CH), jnp.int32),
            pltpu.VMEM((2, _CH, _D), F32),
            pltpu.SemaphoreType.DMA,
            pltpu.SemaphoreType.DMA,
            pltpu.SemaphoreType.DMA,
            pltpu.SemaphoreType.DMA,
        ],
    )(_sc_body)


_R = 1000


def _tc_body(a1_3, a4_3, a2_3, a3_3, d0_3, d1_3, xs, xu, wcs, wcu, wfs, wfu,
             bf, os_ref, ou_ref):
    a1, a4, a2, a3 = a1_3[0], a4_3[0], a2_3[0], a3_3[0]
    deg = d0_3[0, :, :1] + d1_3[0, :, :1]
    inv = 1.0 / jnp.maximum(deg, 1.0)
    s_in = jnp.concatenate([a1, a4], axis=1) * inv
    u_in = jnp.concatenate([a3, a2], axis=1) * inv
    hs = jnp.tanh(
        jnp.dot(s_in, wcs[...], preferred_element_type=F32) + xs[...])
    hu = jnp.maximum(
        jnp.dot(u_in, wcu[...], preferred_element_type=F32) + xu[...], 0.0)
    os_ref[...] = jnp.dot(jnp.concatenate([hs, hu], axis=1), wfs[...],
                          preferred_element_type=F32) + hs
    ou_ref[...] = jnp.dot(jnp.concatenate([jnp.abs(hs), hu], axis=1), wfu[...],
                          preferred_element_type=F32) + bf[...] + hu


def _acc_spec(job):
    return pl.BlockSpec((1, _R, _D), lambda i: (job, i, 0))


_tc_epilogue = pl.pallas_call(
    _tc_body,
    grid=(_N // _R,),
    in_specs=[
        _acc_spec(0), _acc_spec(1), _acc_spec(2), _acc_spec(3),
        _acc_spec(4), _acc_spec(5),
        pl.BlockSpec((_R, _D), lambda i: (i, 0)),
        pl.BlockSpec((_R, _D), lambda i: (i, 0)),
        pl.BlockSpec((2 * _D, _D), lambda i: (0, 0)),
        pl.BlockSpec((2 * _D, _D), lambda i: (0, 0)),
        pl.BlockSpec((2 * _D, _D), lambda i: (0, 0)),
        pl.BlockSpec((2 * _D, _D), lambda i: (0, 0)),
        pl.BlockSpec((1, _D), lambda i: (0, 0)),
    ],
    out_specs=[
        pl.BlockSpec((_R, _D), lambda i: (i, 0)),
        pl.BlockSpec((_R, _D), lambda i: (i, 0)),
    ],
    out_shape=[
        jax.ShapeDtypeStruct((_N, _D), F32),
        jax.ShapeDtypeStruct((_N, _D), F32),
    ],
)


def kernel(x_signed, x_unsigned, edge_index, is_directed,
           W_ss, W_su, W_uu, W_us, Wf_s, Wf_u, bf_u):
    src = edge_index[0]
    dst = edge_index[1]
    dir_i = is_directed.astype(jnp.int32)

    zrow8 = jnp.zeros((8, _D), F32)
    t1 = jnp.concatenate([x_signed, -x_signed, zrow8], axis=0)
    t4 = jnp.concatenate([x_unsigned, -x_unsigned, zrow8], axis=0)
    t2 = jnp.concatenate([jnp.abs(x_signed), zrow8], axis=0)
    t3 = jnp.concatenate([x_unsigned, zrow8], axis=0)
    ta = jnp.concatenate([t1, t4, t2, t3], axis=0)

    npad = _EPAD - _E
    g_sd = jnp.concatenate(
        [src + _N * dir_i, jnp.full((npad,), 2 * _N, jnp.int32)])
    g_u = jnp.concatenate([src, jnp.full((npad,), _N, jnp.int32)])
    gidx = jnp.concatenate([
        g_sd + _O[0], g_sd + _O[1], g_u + _O[2], g_u + _O[3]])
    gidx4 = gidx.reshape(4, _NS, _NCHUNK, _CH)
    dstp = jnp.concatenate([dst, jnp.full((npad,), _N, jnp.int32)])
    dst2 = dstp.reshape(_NS, _NCHUNK, _CH)
    zrows = jnp.zeros((_NPAD, _D), F32)
    ones_h = jnp.ones((_CH, _D), F32)

    outf = _make_sc_agg()(ta, gidx4, dst2, zrows, ones_h)

    new_s, new_u = _tc_epilogue(
        outf, outf, outf, outf, outf, outf, x_signed, x_unsigned,
        jnp.concatenate([W_ss, W_us], axis=0),
        jnp.concatenate([W_uu, W_su], axis=0),
        Wf_s, Wf_u, bf_u.reshape(1, _D))
    return (new_s, new_u)


# speedup vs baseline: 3.4597x; 1.0672x over previous
import functools

import jax
import jax.numpy as jnp
from jax import lax
from jax.experimental import pallas as pl
from jax.experimental.pallas import tpu as pltpu
from jax.experimental.pallas import tpu_sc as plsc

F32 = jnp.float32

_N = 10000
_E = 320000
_D = 128
_NS = 16
_CH = 128
_EPT = 20480
_EPAD = _NS * _EPT
_NCHUNK = _EPT // _CH
_EPT2 = _EPAD // 32
_NCHUNK2 = _EPT2 // _CH
_G = 16
_NPAD = 10240
_RPT = _NPAD // _NS

_O = (0, 2 * _N + 8, 4 * _N + 16, 5 * _N + 24)


def _sc_body(ta, gidx4, dst2, zrows, outf, outd, acc, giv, div, rows, degp,
             sg0, sg1, ss0, ss1):
    c = lax.axis_index("c")
    s = lax.axis_index("s")
    wid = c * _NS + s
    sg = (sg0, sg1)
    ss = (ss0, ss1)
    ones16 = jnp.ones((16,), F32)

    def drain(sem, b):
        pltpu.make_async_copy(ta.at[pl.ds(0, _CH)], rows.at[b], sem).wait()

    def zdeg(i, carry):
        degp[pl.ds(i * 16, 16)] = jnp.zeros((16,), F32)
        return carry

    lax.fori_loop(0, _NPAD // 16, zdeg, 0)

    for j in range(2):
        jj = 2 * c + j
        pltpu.sync_copy(zrows.at[pl.ds(s * _RPT, _RPT)],
                        acc.at[pl.ds(s * _RPT, _RPT)])
        plsc.subcore_barrier()

        def group(grp, carry):
            pltpu.sync_copy(gidx4.at[jj, s, pl.ds(grp * _G, _G)], giv)
            pltpu.sync_copy(dst2.at[s, pl.ds(grp * _G, _G)], div)
            pltpu.async_copy(ta.at[giv.at[0]], rows.at[0], sg[0])
            for i in range(_G):
                b = i % 2
                nb = 1 - b
                if i < _G - 1:
                    if i >= 1:
                        drain(ss[nb], nb)
                    pltpu.async_copy(ta.at[giv.at[i + 1]], rows.at[nb],
                                     sg[nb])
                if j == 0:
                    for l in range(_CH // 16):
                        plsc.addupdate_scatter(
                            degp, [div[i, pl.ds(l * 16, 16)]], ones16)
                drain(sg[b], b)
                pltpu.async_copy(rows.at[b], acc.at[div.at[i]], ss[b],
                                 add=True)
            drain(ss[0], 0)
            drain(ss[1], 1)
            return carry

        lax.fori_loop(0, _NCHUNK // _G, group, 0)

        plsc.subcore_barrier()
        pltpu.sync_copy(acc.at[pl.ds(s * _RPT, _RPT)],
                        outf.at[jj, pl.ds(s * _RPT, _RPT)])
        if j == 0:
            pltpu.sync_copy(degp, outd.at[wid])
        plsc.subcore_barrier()


def _make_sc_agg():
    return functools.partial(
        pl.kernel,
        mesh=plsc.VectorSubcoreMesh(core_axis_name="c", subcore_axis_name="s"),
        compiler_params=pltpu.CompilerParams(needs_layout_passes=False),
        out_type=[jax.ShapeDtypeStruct((4, _NPAD, _D), F32),
                  jax.ShapeDtypeStruct((2 * _NS, _NPAD), F32)],
        scratch_types=[
            pltpu.VMEM_SHARED((_NPAD, _D), F32),
            pltpu.VMEM((_G, _CH), jnp.int32),
            pltpu.VMEM((_G, _CH), jnp.int32),
            pltpu.VMEM((2, _CH, _D), F32),
            pltpu.VMEM((_NPAD,), F32),
            pltpu.SemaphoreType.DMA,
            pltpu.SemaphoreType.DMA,
            pltpu.SemaphoreType.DMA,
            pltpu.SemaphoreType.DMA,
        ],
    )(_sc_body)


_R = 1024


def _tc_body(a1_3, a4_3, a2_3, a3_3, dp, xs, xu, wcs, wcu, wfs, wfu,
             bf, os_ref, ou_ref):
    a1, a4, a2, a3 = a1_3[0], a4_3[0], a2_3[0], a3_3[0]
    deg2 = jnp.sum(dp[0], axis=0)[:, None]
    inv = 2.0 / jnp.maximum(deg2, 2.0)
    s_in = jnp.concatenate([a1, a4], axis=1) * inv
    u_in = jnp.concatenate([a3, a2], axis=1) * inv
    hs = jnp.tanh(
        jnp.dot(s_in, wcs[...], preferred_element_type=F32) + xs[...])
    hu = jnp.maximum(
        jnp.dot(u_in, wcu[...], preferred_element_type=F32) + xu[...], 0.0)
    os_ref[...] = jnp.dot(jnp.concatenate([hs, hu], axis=1), wfs[...],
                          preferred_element_type=F32) + hs
    ou_ref[...] = jnp.dot(jnp.concatenate([jnp.abs(hs), hu], axis=1), wfu[...],
                          preferred_element_type=F32) + bf[...] + hu


def _acc_spec(job):
    return pl.BlockSpec((1, _R, _D), lambda i: (job, i, 0))


_tc_epilogue = pl.pallas_call(
    _tc_body,
    grid=(_NPAD // _R,),
    in_specs=[
        _acc_spec(0), _acc_spec(1), _acc_spec(2), _acc_spec(3),
        pl.BlockSpec((1, 2 * _NS, _R), lambda i: (i, 0, 0)),
        pl.BlockSpec((_R, _D), lambda i: (i, 0)),
        pl.BlockSpec((_R, _D), lambda i: (i, 0)),
        pl.BlockSpec((2 * _D, _D), lambda i: (0, 0)),
        pl.BlockSpec((2 * _D, _D), lambda i: (0, 0)),
        pl.BlockSpec((2 * _D, _D), lambda i: (0, 0)),
        pl.BlockSpec((2 * _D, _D), lambda i: (0, 0)),
        pl.BlockSpec((1, _D), lambda i: (0, 0)),
    ],
    out_specs=[
        pl.BlockSpec((_R, _D), lambda i: (i, 0)),
        pl.BlockSpec((_R, _D), lambda i: (i, 0)),
    ],
    out_shape=[
        jax.ShapeDtypeStruct((_NPAD, _D), F32),
        jax.ShapeDtypeStruct((_NPAD, _D), F32),
    ],
)


def kernel(x_signed, x_unsigned, edge_index, is_directed,
           W_ss, W_su, W_uu, W_us, Wf_s, Wf_u, bf_u):
    src = edge_index[0]
    dst = edge_index[1]
    dir_i = is_directed.astype(jnp.int32)

    zrow8 = jnp.zeros((8, _D), F32)
    t1 = jnp.concatenate([x_signed, -x_signed, zrow8], axis=0)
    t4 = jnp.concatenate([x_unsigned, -x_unsigned, zrow8], axis=0)
    t2 = jnp.concatenate([jnp.abs(x_signed), zrow8], axis=0)
    t3 = jnp.concatenate([x_unsigned, zrow8], axis=0)
    ta = jnp.concatenate([t1, t4, t2, t3], axis=0)

    npad = _EPAD - _E
    g_sd = jnp.concatenate(
        [src + _N * dir_i, jnp.full((npad,), 2 * _N, jnp.int32)])
    g_u = jnp.concatenate([src, jnp.full((npad,), _N, jnp.int32)])
    gidx = jnp.concatenate([
        g_sd + _O[0], g_sd + _O[1], g_u + _O[2], g_u + _O[3]])
    gidx4 = gidx.reshape(4, _NS, _NCHUNK, _CH)
    dstp = jnp.concatenate([dst, jnp.full((npad,), _N, jnp.int32)])
    dst2 = dstp.reshape(_NS, _NCHUNK, _CH)
    zrows = jnp.zeros((_NPAD, _D), F32)

    outf, outd = _make_sc_agg()(ta, gidx4, dst2, zrows)
    outd3 = outd.reshape(2 * _NS, _NPAD // _R, _R).transpose(1, 0, 2)
    zn = jnp.zeros((_NPAD - _N, _D), F32)

    new_s, new_u = _tc_epilogue(
        outf, outf, outf, outf, outd3,
        jnp.concatenate([x_signed, zn], axis=0),
        jnp.concatenate([x_unsigned, zn], axis=0),
        jnp.concatenate([W_ss, W_us], axis=0),
        jnp.concatenate([W_uu, W_su], axis=0),
        Wf_s, Wf_u, bf_u.reshape(1, _D))
    return (new_s[:_N], new_u[:_N])


# speedup vs baseline: 3.5795x; 1.0346x over previous
import functools

import jax
import jax.numpy as jnp
from jax import lax
from jax.experimental import pallas as pl
from jax.experimental.pallas import tpu as pltpu
from jax.experimental.pallas import tpu_sc as plsc

F32 = jnp.float32

_N = 10000
_E = 320000
_D = 128
_NS = 16
_CH = 128
_EPT = 20480
_EPAD = _NS * _EPT
_NCHUNK = _EPT // _CH
_G = 8
_NG = _NCHUNK // _G
_NPAD = 10240
_RPT = _NPAD // _NS

_O = (0, 2 * _N + 8, 4 * _N + 16, 5 * _N + 24)


def _sc_body(ta, gidx4, dst2, zrows, outf, outd, acc, giv, div, rows, degp,
             sg0, sg1, ss0, ss1, si):
    c = lax.axis_index("c")
    s = lax.axis_index("s")
    wid = c * _NS + s
    sg = (sg0, sg1)
    ss = (ss0, ss1)
    ones16 = jnp.ones((16,), F32)

    def drain(sem, b):
        pltpu.make_async_copy(ta.at[pl.ds(0, _CH)], rows.at[b], sem).wait()

    def zdeg(i, carry):
        degp[pl.ds(i * 16, 16)] = jnp.zeros((16,), F32)
        return carry

    lax.fori_loop(0, _NPAD // 16, zdeg, 0)

    for j in range(2):
        jj = 2 * c + j
        pltpu.sync_copy(zrows.at[pl.ds(s * _RPT, _RPT)],
                        acc.at[pl.ds(s * _RPT, _RPT)])
        plsc.subcore_barrier()

        pltpu.sync_copy(gidx4.at[jj, s, pl.ds(0, _G)], giv.at[0])
        pltpu.sync_copy(dst2.at[s, pl.ds(0, _G)], div.at[0])
        pltpu.async_copy(ta.at[giv.at[0, 0]], rows.at[0], sg[0])

        def group(grp, carry):
            p = grp % 2
            np_ = 1 - p
            nxt = ((grp + 1) % _NG) * _G
            for i in range(_G):
                b = i % 2
                nb = 1 - b
                if i == 0:
                    @pl.when(grp > 0)
                    def _():
                        drain(ss[nb], nb)
                else:
                    drain(ss[nb], nb)
                if i == 2:
                    pltpu.async_copy(gidx4.at[jj, s, pl.ds(nxt, _G)],
                                     giv.at[np_], si)
                    pltpu.async_copy(dst2.at[s, pl.ds(nxt, _G)],
                                     div.at[np_], si)
                if i < _G - 1:
                    pltpu.async_copy(ta.at[giv.at[p, i + 1]], rows.at[nb],
                                     sg[nb])
                else:
                    pltpu.make_async_copy(gidx4.at[jj, s, pl.ds(0, _G)],
                                          giv.at[0], si).wait()
                    pltpu.make_async_copy(gidx4.at[jj, s, pl.ds(0, _G)],
                                          giv.at[0], si).wait()
                    pltpu.async_copy(ta.at[giv.at[np_, 0]], rows.at[nb],
                                     sg[nb])
                if j == 0:
                    for l in range(_CH // 16):
                        plsc.addupdate_scatter(
                            degp, [div[p, i, pl.ds(l * 16, 16)]], ones16)
                drain(sg[b], b)
                pltpu.async_copy(rows.at[b], acc.at[div.at[p, i]], ss[b],
                                 add=True)
            return carry

        lax.fori_loop(0, _NG, group, 0)
        drain(sg[0], 0)
        drain(ss[(_NCHUNK - 1) % 2], (_NCHUNK - 1) % 2)

        plsc.subcore_barrier()
        pltpu.sync_copy(acc.at[pl.ds(s * _RPT, _RPT)],
                        outf.at[jj, pl.ds(s * _RPT, _RPT)])
        if j == 0:
            pltpu.sync_copy(degp, outd.at[wid])
        plsc.subcore_barrier()


def _make_sc_agg():
    return functools.partial(
        pl.kernel,
        mesh=plsc.VectorSubcoreMesh(core_axis_name="c", subcore_axis_name="s"),
        compiler_params=pltpu.CompilerParams(needs_layout_passes=False),
        out_type=[jax.ShapeDtypeStruct((4, _NPAD, _D), F32),
                  jax.ShapeDtypeStruct((2 * _NS, _NPAD), F32)],
        scratch_types=[
            pltpu.VMEM_SHARED((_NPAD, _D), F32),
            pltpu.VMEM((2, _G, _CH), jnp.int32),
            pltpu.VMEM((2, _G, _CH), jnp.int32),
            pltpu.VMEM((2, _CH, _D), F32),
            pltpu.VMEM((_NPAD,), F32),
            pltpu.SemaphoreType.DMA,
            pltpu.SemaphoreType.DMA,
            pltpu.SemaphoreType.DMA,
            pltpu.SemaphoreType.DMA,
            pltpu.SemaphoreType.DMA,
        ],
    )(_sc_body)


_R = 1024


def _tc_body(a1_3, a4_3, a2_3, a3_3, dp, xs, xu, wcs, wcu, wfs, wfu,
             bf, os_ref, ou_ref):
    a1, a4, a2, a3 = a1_3[0], a4_3[0], a2_3[0], a3_3[0]
    deg2 = jnp.sum(dp[0], axis=0)[:, None]
    inv = 2.0 / jnp.maximum(deg2, 2.0)
    s_in = jnp.concatenate([a1, a4], axis=1) * inv
    u_in = jnp.concatenate([a3, a2], axis=1) * inv
    hs = jnp.tanh(
        jnp.dot(s_in, wcs[...], preferred_element_type=F32) + xs[...])
    hu = jnp.maximum(
        jnp.dot(u_in, wcu[...], preferred_element_type=F32) + xu[...], 0.0)
    os_ref[...] = jnp.dot(jnp.concatenate([hs, hu], axis=1), wfs[...],
                          preferred_element_type=F32) + hs
    ou_ref[...] = jnp.dot(jnp.concatenate([jnp.abs(hs), hu], axis=1), wfu[...],
                          preferred_element_type=F32) + bf[...] + hu


def _acc_spec(job):
    return pl.BlockSpec((1, _R, _D), lambda i: (job, i, 0))


_tc_epilogue = pl.pallas_call(
    _tc_body,
    grid=(_NPAD // _R,),
    in_specs=[
        _acc_spec(0), _acc_spec(1), _acc_spec(2), _acc_spec(3),
        pl.BlockSpec((1, 2 * _NS, _R), lambda i: (i, 0, 0)),
        pl.BlockSpec((_R, _D), lambda i: (i, 0)),
        pl.BlockSpec((_R, _D), lambda i: (i, 0)),
        pl.BlockSpec((2 * _D, _D), lambda i: (0, 0)),
        pl.BlockSpec((2 * _D, _D), lambda i: (0, 0)),
        pl.BlockSpec((2 * _D, _D), lambda i: (0, 0)),
        pl.BlockSpec((2 * _D, _D), lambda i: (0, 0)),
        pl.BlockSpec((1, _D), lambda i: (0, 0)),
    ],
    out_specs=[
        pl.BlockSpec((_R, _D), lambda i: (i, 0)),
        pl.BlockSpec((_R, _D), lambda i: (i, 0)),
    ],
    out_shape=[
        jax.ShapeDtypeStruct((_NPAD, _D), F32),
        jax.ShapeDtypeStruct((_NPAD, _D), F32),
    ],
)


def kernel(x_signed, x_unsigned, edge_index, is_directed,
           W_ss, W_su, W_uu, W_us, Wf_s, Wf_u, bf_u):
    src = edge_index[0]
    dst = edge_index[1]
    dir_i = is_directed.astype(jnp.int32)

    zrow8 = jnp.zeros((8, _D), F32)
    t1 = jnp.concatenate([x_signed, -x_signed, zrow8], axis=0)
    t4 = jnp.concatenate([x_unsigned, -x_unsigned, zrow8], axis=0)
    t2 = jnp.concatenate([jnp.abs(x_signed), zrow8], axis=0)
    t3 = jnp.concatenate([x_unsigned, zrow8], axis=0)
    ta = jnp.concatenate([t1, t4, t2, t3], axis=0)

    npad = _EPAD - _E
    g_sd = jnp.concatenate(
        [src + _N * dir_i, jnp.full((npad,), 2 * _N, jnp.int32)])
    g_u = jnp.concatenate([src, jnp.full((npad,), _N, jnp.int32)])
    gidx = jnp.concatenate([
        g_sd + _O[0], g_sd + _O[1], g_u + _O[2], g_u + _O[3]])
    gidx4 = gidx.reshape(4, _NS, _NCHUNK, _CH)
    dstp = jnp.concatenate([dst, jnp.full((npad,), _N, jnp.int32)])
    dst2 = dstp.reshape(_NS, _NCHUNK, _CH)
    zrows = jnp.zeros((_NPAD, _D), F32)

    outf, outd = _make_sc_agg()(ta, gidx4, dst2, zrows)
    outd3 = outd.reshape(2 * _NS, _NPAD // _R, _R).transpose(1, 0, 2)
    zn = jnp.zeros((_NPAD - _N, _D), F32)

    new_s, new_u = _tc_epilogue(
        outf, outf, outf, outf, outd3,
        jnp.concatenate([x_signed, zn], axis=0),
        jnp.concatenate([x_unsigned, zn], axis=0),
        jnp.concatenate([W_ss, W_us], axis=0),
        jnp.concatenate([W_uu, W_su], axis=0),
        Wf_s, Wf_u, bf_u.reshape(1, _D))
    return (new_s[:_N], new_u[:_N])
